# baseline (speedup 1.0000x reference)
import functools

import jax
import jax.numpy as jnp
from jax import lax
from jax.experimental import pallas as pl
from jax.experimental.pallas import tpu as pltpu
from jax.experimental.pallas import tpu_sc as plsc

_L = 16
_NB = 3


def _sc_body(B, T, S, D, TPW, TC, x_hbm, emb_hbm, out_hbm, *scratch):
    wid = lax.axis_index("s") * 2 + lax.axis_index("c")
    t0 = wid * TPW
    cpb = TPW // TC
    nch = B * cpb
    shift = cpb.bit_length() - 1
    xbs, ebs = scratch[0:_NB], scratch[_NB:2 * _NB]
    sxs, ses, sos = (scratch[2 * _NB:3 * _NB], scratch[3 * _NB:4 * _NB],
                     scratch[4 * _NB:5 * _NB])

    def coords(i):
        b = lax.shift_right_logical(i, shift)
        c = lax.bitwise_and(i, cpb - 1)
        return b, t0 + c * TC

    def start_in(i, slot):
        b, tb = coords(i)
        pltpu.make_async_copy(
            x_hbm.at[b, pl.ds(tb, TC)], xbs[slot], sxs[slot]).start()
        pltpu.make_async_copy(
            emb_hbm.at[pl.ds(tb, TC)], ebs[slot], ses[slot]).start()

    def wait_in(slot):
        pltpu.make_async_copy(
            x_hbm.at[0, pl.ds(0, TC)], xbs[slot], sxs[slot]).wait()
        pltpu.make_async_copy(
            emb_hbm.at[pl.ds(0, TC)], ebs[slot], ses[slot]).wait()

    def start_out(i, slot):
        b, tb = coords(i)
        pltpu.make_async_copy(
            xbs[slot], out_hbm.at[b, pl.ds(tb, TC)], sos[slot]).start()

    def wait_out(slot):
        pltpu.make_async_copy(
            xbs[slot], out_hbm.at[0, pl.ds(0, TC)], sos[slot]).wait()

    def compute(slot):
        xb, eb = xbs[slot], ebs[slot]
        for t in range(TC):
            @plsc.parallel_loop(0, D // _L, unroll=8)
            def _body(k):
                ev = eb[t, pl.ds(k * _L, _L)]
                for s_ in range(S):
                    xb[t, s_, pl.ds(k * _L, _L)] = (
                        xb[t, s_, pl.ds(k * _L, _L)] + ev)

    for j in range(_NB - 1):
        start_in(jnp.int32(j), j)

    n_rings = (nch - (_NB - 1)) // _NB
    assert n_rings * _NB + (_NB - 1) == nch

    def ring(iq, carry):
        for j in range(_NB):
            i = iq * _NB + j
            tgt = (j + _NB - 1) % _NB

            @pl.when(i >= 1)
            def _drain():
                wait_out(tgt)

            start_in(i + _NB - 1, tgt)
            wait_in(j)
            compute(j)
            start_out(i, j)
        return carry

    lax.fori_loop(0, n_rings, ring, 0)

    for r in range(_NB - 1):
        i = n_rings * _NB + r
        j = i % _NB
        wait_in(j)
        compute(j)
        start_out(jnp.int32(i), j)

    for j in range(_NB):
        wait_out(j)


def kernel(x, pos_embedding):
    B, T, S, D = x.shape
    NW = 32
    TPW = T // NW
    TC = 8

    mesh = plsc.VectorSubcoreMesh(core_axis_name="c", subcore_axis_name="s")
    run = pl.kernel(
        functools.partial(_sc_body, B, T, S, D, TPW, TC),
        out_type=jax.ShapeDtypeStruct((B, T, S, D), jnp.float32),
        mesh=mesh,
        scratch_types=(
            [pltpu.VMEM((TC, S, D), jnp.float32) for _ in range(_NB)]
            + [pltpu.VMEM((TC, D), jnp.float32) for _ in range(_NB)]
            + [pltpu.SemaphoreType.DMA for _ in range(3 * _NB)]
        ),
    )
    return run(x, pos_embedding)
